# baseline (speedup 1.0000x reference)
import functools

import jax
import jax.numpy as jnp
from jax import lax
from jax.experimental import pallas as pl
from jax.experimental.pallas import tpu as pltpu
from jax.experimental.pallas import tpu_sc as plsc

IN_DIM = 128
ATTN_DIM = 64
N_NODE = 10000
E_TOTAL = 320000
B = 32
R = 200
TBL = 256
SLOPE = (1.0 / 8.0 + 1.0 / 3.0) / 2.0

NC = 2
NS = 16
NW = NC * NS
EPW = E_TOTAL // NW
CHUNK = 40
NCHUNK = EPW // CHUNK
CPS = 10
SUPC = CPS * CHUNK
NSUPER = NCHUNK // CPS
ROWS_PER_TILE = 624
ROWS_TAIL = N_NODE - NS * ROWS_PER_TILE


def _rrelu(x):
    return jnp.where(x >= 0, x, x * SLOPE)


def _hsum16(v):
    lanes = lax.broadcasted_iota(jnp.int32, (16,), 0)
    dnums = lax.GatherDimensionNumbers(
        offset_dims=(), collapsed_slice_dims=(0,), start_index_map=(0,))
    for s in (8, 4, 2, 1):
        perm = lax.reshape(lanes ^ s, (16, 1))
        v = v + lax.gather(v, perm, dnums, (1,),
                           mode=lax.GatherScatterMode.PROMISE_IN_BOUNDS)
    return v


def _qsel_body(oh_ref, re_ref, o_ref):
    o_ref[...] = jnp.dot(oh_ref[...], re_ref[...],
                         preferred_element_type=jnp.float32,
                         precision=lax.Precision.HIGHEST)


def _qsel(onehot, rel_emb):
    n = B * (R + 1)
    return pl.pallas_call(
        _qsel_body,
        grid=(1,),
        in_specs=[pl.BlockSpec((B, n), lambda i: (0, 0)),
                  pl.BlockSpec((n, IN_DIM), lambda i: (0, 0))],
        out_specs=pl.BlockSpec((B, IN_DIM), lambda i: (0, 0)),
        out_shape=jax.ShapeDtypeStruct((B, IN_DIM), jnp.float32),
    )(onehot, rel_emb)


def _proj_node_body(h_ref, ws_ref, o_ref):
    o_ref[...] = jnp.dot(h_ref[...], ws_ref[...],
                         preferred_element_type=jnp.float32,
                         precision=lax.Precision.HIGHEST)


def _proj_node(hidden, Ws):
    return pl.pallas_call(
        _proj_node_body,
        grid=(10,),
        in_specs=[pl.BlockSpec((1000, IN_DIM), lambda i: (i, 0)),
                  pl.BlockSpec((IN_DIM, ATTN_DIM), lambda i: (0, 0))],
        out_specs=pl.BlockSpec((1000, ATTN_DIM), lambda i: (i, 0)),
        out_shape=jax.ShapeDtypeStruct((N_NODE, ATTN_DIM), jnp.float32),
    )(hidden, Ws)


def _proj_rel_body(r_ref, q_ref, wr_ref, wqr_ref, bqr_ref, o_ref):
    o_ref[...] = (
        jnp.dot(r_ref[...], wr_ref[...],
                preferred_element_type=jnp.float32,
                precision=lax.Precision.HIGHEST)
        + jnp.dot(q_ref[...], wqr_ref[...],
                  preferred_element_type=jnp.float32,
                  precision=lax.Precision.HIGHEST)
        + bqr_ref[...]
    )


def _proj_rel(rel_emb, q_rep, Wr, Wqr, bqr_row):
    n = B * (R + 1)
    return pl.pallas_call(
        _proj_rel_body,
        grid=(4,),
        in_specs=[pl.BlockSpec((1608, IN_DIM), lambda i: (i, 0)),
                  pl.BlockSpec((1608, IN_DIM), lambda i: (i, 0)),
                  pl.BlockSpec((IN_DIM, ATTN_DIM), lambda i: (0, 0)),
                  pl.BlockSpec((IN_DIM, ATTN_DIM), lambda i: (0, 0)),
                  pl.BlockSpec((1, ATTN_DIM), lambda i: (0, 0))],
        out_specs=pl.BlockSpec((1608, ATTN_DIM), lambda i: (i, 0)),
        out_shape=jax.ShapeDtypeStruct((n, ATTN_DIM), jnp.float32),
    )(rel_emb, q_rep, Wr, Wqr, bqr_row)


_sc_mesh = plsc.VectorSubcoreMesh(core_axis_name="c", subcore_axis_name="s",
                                  num_cores=NC, num_subcores=NS)


@functools.partial(
    pl.kernel,
    out_type=jax.ShapeDtypeStruct((NC, N_NODE, IN_DIM), jnp.float32),
    mesh=_sc_mesh,
    scratch_types=[
        pltpu.VMEM((4 * SUPC,), jnp.int32),
        pltpu.VMEM((2 * SUPC,), jnp.int32),
        pltpu.VMEM((2 * CHUNK, TBL), jnp.float32),
        pltpu.VMEM((2 * CHUNK, TBL), jnp.float32),
        pltpu.VMEM((CHUNK, IN_DIM), jnp.float32),
        pltpu.VMEM((ATTN_DIM,), jnp.float32),
        pltpu.VMEM((16,), jnp.float32),
        pltpu.VMEM_SHARED((N_NODE, IN_DIM), jnp.float32),
        pltpu.SemaphoreType.DMA,
        pltpu.SemaphoreType.DMA,
        pltpu.SemaphoreType.DMA,
        pltpu.SemaphoreType.DMA,
        pltpu.SemaphoreType.DMA,
    ],
)
def _sc_edges(tbl_hbm, fidx_hbm, obj_hbm, w_hbm, b_hbm,
              zeros_hbm, out_hbm,
              ifu, iobj, buf0, buf1, msg, wbuf, bbuf, acc,
              sem_g0, sem_g1, sem_ia, sem_ib, sem_sc):
    cid = lax.axis_index("c")
    sid = lax.axis_index("s")
    wid = sid * NC + cid

    r0 = sid * ROWS_PER_TILE
    pltpu.sync_copy(zeros_hbm.at[pl.ds(r0, ROWS_PER_TILE)],
                    acc.at[pl.ds(r0, ROWS_PER_TILE)])

    @pl.when(sid == NS - 1)
    def _zero_tail():
        pltpu.sync_copy(zeros_hbm.at[pl.ds(NS * ROWS_PER_TILE, ROWS_TAIL)],
                        acc.at[pl.ds(NS * ROWS_PER_TILE, ROWS_TAIL)])

    pltpu.sync_copy(w_hbm, wbuf)
    pltpu.sync_copy(b_hbm, bbuf)
    plsc.subcore_barrier()

    base_w = wid * EPW
    bufs = ((buf0, sem_g0), (buf1, sem_g1))

    def load_idx(si):
        off = (si % 2) * SUPC
        base = base_w + si * SUPC
        pltpu.async_copy(fidx_hbm.at[pl.ds(2 * base, 2 * SUPC)],
                         ifu.at[pl.ds(2 * off, 2 * SUPC)], sem_ia)
        pltpu.async_copy(obj_hbm.at[pl.ds(base, SUPC)],
                         iobj.at[pl.ds(off, SUPC)], sem_ib)

    def wait_idx(si):
        off = (si % 2) * SUPC
        base = base_w + si * SUPC
        pltpu.make_async_copy(fidx_hbm.at[pl.ds(2 * base, 2 * SUPC)],
                              ifu.at[pl.ds(2 * off, 2 * SUPC)], sem_ia).wait()
        pltpu.make_async_copy(obj_hbm.at[pl.ds(base, SUPC)],
                              iobj.at[pl.ds(off, SUPC)], sem_ib).wait()

    def gather(idx_off, par):
        buf, sem_g = bufs[par]
        pltpu.async_copy(tbl_hbm.at[ifu.at[pl.ds(2 * idx_off, 2 * CHUNK)]],
                         buf, sem_g)

    def finish(idx_off, par):
        buf, sem_g = bufs[par]
        pltpu.make_async_copy(tbl_hbm.at[ifu.at[pl.ds(2 * idx_off, 2 * CHUNK)]],
                              buf, sem_g).wait()

        pltpu.make_async_copy(
            msg, acc.at[iobj.at[pl.ds(idx_off, CHUNK)]], sem_sc).wait()

        w_vecs = [wbuf[pl.ds(16 * j, 16)] for j in range(4)]
        bvec = bbuf[...]

        @plsc.parallel_loop(0, CHUNK, unroll=8)
        def edge_body(e):
            u = None
            for j in range(4):
                pre = (buf[2 * e, pl.ds(16 * j, 16)]
                       + buf[2 * e + 1, pl.ds(16 * j, 16)])
                t = _rrelu(pre) * w_vecs[j]
                u = t if u is None else u + t
            dot = _hsum16(u)
            av = 1.0 / (1.0 + jnp.exp(-(dot + bvec)))
            for j in range(8):
                m = (buf[2 * e, pl.ds(ATTN_DIM + 16 * j, 16)]
                     + buf[2 * e + 1, pl.ds(ATTN_DIM + 16 * j, 16)]) * av
                msg[e, pl.ds(16 * j, 16)] = m

        pltpu.async_copy(msg, acc.at[iobj.at[pl.ds(idx_off, CHUNK)]], sem_sc,
                         add=True)

    load_idx(0)
    wait_idx(0)
    gather(0, 0)
    pltpu.sync_copy(zeros_hbm.at[pl.ds(0, CHUNK)], msg)
    pltpu.async_copy(msg, acc.at[iobj.at[pl.ds(0, CHUNK)]], sem_sc, add=True)

    def super_body(si, carry):
        off = (si % 2) * SUPC

        @pl.when(si < NSUPER - 1)
        def _prefetch_idx():
            load_idx(si + 1)

        def pair_body(j, c2):
            o0 = off + (2 * j) * CHUNK
            gather(o0 + CHUNK, 1)
            finish(o0, 0)

            @pl.when(j < CPS // 2 - 1)
            def _next_even():
                gather(o0 + 2 * CHUNK, 0)

            finish(o0 + CHUNK, 1)
            return c2

        lax.fori_loop(0, CPS // 2, pair_body, 0)

        @pl.when(si < NSUPER - 1)
        def _start_next_super():
            wait_idx(si + 1)
            gather(((si + 1) % 2) * SUPC, 0)

        return carry

    lax.fori_loop(0, NSUPER, super_body, 0)
    pltpu.make_async_copy(
        msg, acc.at[iobj.at[pl.ds(0, CHUNK)]], sem_sc).wait()

    plsc.subcore_barrier()
    pltpu.sync_copy(acc.at[pl.ds(r0, ROWS_PER_TILE)],
                    out_hbm.at[cid, pl.ds(r0, ROWS_PER_TILE)])

    @pl.when(sid == NS - 1)
    def _drain_tail():
        pltpu.sync_copy(acc.at[pl.ds(NS * ROWS_PER_TILE, ROWS_TAIL)],
                        out_hbm.at[cid, pl.ds(NS * ROWS_PER_TILE, ROWS_TAIL)])


def _final_body(a0_ref, a1_ref, wh_ref, o_ref):
    acc = a0_ref[...] + a1_ref[...]
    o_ref[...] = _rrelu(jnp.dot(acc, wh_ref[...],
                                preferred_element_type=jnp.float32,
                                precision=lax.Precision.HIGHEST))


def _final(acc0, acc1, W_h):
    return pl.pallas_call(
        _final_body,
        grid=(10,),
        in_specs=[pl.BlockSpec((1000, IN_DIM), lambda i: (i, 0)),
                  pl.BlockSpec((1000, IN_DIM), lambda i: (i, 0)),
                  pl.BlockSpec((IN_DIM, IN_DIM), lambda i: (0, 0))],
        out_specs=pl.BlockSpec((1000, IN_DIM), lambda i: (i, 0)),
        out_shape=jax.ShapeDtypeStruct((N_NODE, IN_DIM), jnp.float32),
    )(acc0, acc1, W_h)


def kernel(hidden, rel_embeddings, q_rel, batch_idx, rel, sub, obj,
           Ws, Wr, Wqr, bqr, w_alpha, b_alpha, W_h):
    q_idx = q_rel.astype(jnp.int32) + jnp.arange(B, dtype=jnp.int32) * (R + 1)
    onehot = (q_idx[:, None]
              == jnp.arange(B * (R + 1), dtype=jnp.int32)[None, :]
              ).astype(jnp.float32)
    q_sel = _qsel(onehot, rel_embeddings)
    q_rep = jnp.repeat(q_sel, R + 1, axis=0)

    proj_s = _proj_node(hidden, Ws)
    proj_r = _proj_rel(rel_embeddings, q_rep, Wr, Wqr,
                       bqr.reshape(1, ATTN_DIM))

    pad_s = jnp.zeros((N_NODE, TBL - ATTN_DIM - IN_DIM), jnp.float32)
    pad_r = jnp.zeros((B * (R + 1), TBL - ATTN_DIM - IN_DIM), jnp.float32)
    s_tbl = jnp.concatenate([proj_s, hidden, pad_s], axis=1)
    rt_tbl = jnp.concatenate([proj_r, rel_embeddings, pad_r], axis=1)

    tbl = jnp.concatenate([s_tbl, rt_tbl], axis=0)
    eidx = (rel + batch_idx * (R + 1)).astype(jnp.int32) + N_NODE
    fidx = jnp.stack([sub.astype(jnp.int32), eidx], axis=1).reshape(-1)
    acc = _sc_edges(tbl, fidx, obj.astype(jnp.int32), w_alpha[:, 0],
                    jnp.full((16,), b_alpha[0], jnp.float32),
                    jnp.zeros((N_NODE, IN_DIM), jnp.float32))

    return _final(acc[0], acc[1], W_h)


# speedup vs baseline: 1.4806x; 1.4806x over previous
import functools

import jax
import jax.numpy as jnp
from jax import lax
from jax.experimental import pallas as pl
from jax.experimental.pallas import tpu as pltpu
from jax.experimental.pallas import tpu_sc as plsc

IN_DIM = 128
ATTN_DIM = 64
N_NODE = 10000
E_TOTAL = 320000
B = 32
R = 200
TBL = 256
SLOPE = (1.0 / 8.0 + 1.0 / 3.0) / 2.0

NC = 2
NS = 16
NW = NC * NS
EPW = E_TOTAL // NW
CHUNK = 40
NCHUNK = EPW // CHUNK
CPS = 10
SUPC = CPS * CHUNK
NSUPER = NCHUNK // CPS
ROWS_PER_TILE = 624
ROWS_TAIL = N_NODE - NS * ROWS_PER_TILE


def _rrelu(x):
    return jnp.where(x >= 0, x, x * SLOPE)


def _hsum16(v):
    lanes = lax.broadcasted_iota(jnp.int32, (16,), 0)
    dnums = lax.GatherDimensionNumbers(
        offset_dims=(), collapsed_slice_dims=(0,), start_index_map=(0,))
    for s in (8, 4, 2, 1):
        perm = lax.reshape(lanes ^ s, (16, 1))
        v = v + lax.gather(v, perm, dnums, (1,),
                           mode=lax.GatherScatterMode.PROMISE_IN_BOUNDS)
    return v


def _qsel_body(oh_ref, re_ref, o_ref):
    o_ref[...] = jnp.dot(oh_ref[...], re_ref[...],
                         preferred_element_type=jnp.float32,
                         precision=lax.Precision.HIGHEST)


def _qsel(onehot, rel_emb):
    n = B * (R + 1)
    return pl.pallas_call(
        _qsel_body,
        grid=(1,),
        in_specs=[pl.BlockSpec((B, n), lambda i: (0, 0)),
                  pl.BlockSpec((n, IN_DIM), lambda i: (0, 0))],
        out_specs=pl.BlockSpec((B, IN_DIM), lambda i: (0, 0)),
        out_shape=jax.ShapeDtypeStruct((B, IN_DIM), jnp.float32),
    )(onehot, rel_emb)


def _proj_node_body(h_ref, ws_ref, o_ref):
    o_ref[...] = jnp.dot(h_ref[...], ws_ref[...],
                         preferred_element_type=jnp.float32,
                         precision=lax.Precision.HIGHEST)


def _proj_node(hidden, Ws):
    return pl.pallas_call(
        _proj_node_body,
        grid=(10,),
        in_specs=[pl.BlockSpec((1000, IN_DIM), lambda i: (i, 0)),
                  pl.BlockSpec((IN_DIM, ATTN_DIM), lambda i: (0, 0))],
        out_specs=pl.BlockSpec((1000, ATTN_DIM), lambda i: (i, 0)),
        out_shape=jax.ShapeDtypeStruct((N_NODE, ATTN_DIM), jnp.float32),
    )(hidden, Ws)


def _proj_rel_body(r_ref, q_ref, wr_ref, wqr_ref, bqr_ref, o_ref):
    o_ref[...] = (
        jnp.dot(r_ref[...], wr_ref[...],
                preferred_element_type=jnp.float32,
                precision=lax.Precision.HIGHEST)
        + jnp.dot(q_ref[...], wqr_ref[...],
                  preferred_element_type=jnp.float32,
                  precision=lax.Precision.HIGHEST)
        + bqr_ref[...]
    )


def _proj_rel(rel_emb, q_rep, Wr, Wqr, bqr_row):
    n = B * (R + 1)
    return pl.pallas_call(
        _proj_rel_body,
        grid=(4,),
        in_specs=[pl.BlockSpec((1608, IN_DIM), lambda i: (i, 0)),
                  pl.BlockSpec((1608, IN_DIM), lambda i: (i, 0)),
                  pl.BlockSpec((IN_DIM, ATTN_DIM), lambda i: (0, 0)),
                  pl.BlockSpec((IN_DIM, ATTN_DIM), lambda i: (0, 0)),
                  pl.BlockSpec((1, ATTN_DIM), lambda i: (0, 0))],
        out_specs=pl.BlockSpec((1608, ATTN_DIM), lambda i: (i, 0)),
        out_shape=jax.ShapeDtypeStruct((n, ATTN_DIM), jnp.float32),
    )(rel_emb, q_rep, Wr, Wqr, bqr_row)


_sc_mesh = plsc.VectorSubcoreMesh(core_axis_name="c", subcore_axis_name="s",
                                  num_cores=NC, num_subcores=NS)


@functools.partial(
    pl.kernel,
    out_type=jax.ShapeDtypeStruct((NC, N_NODE, IN_DIM), jnp.float32),
    mesh=_sc_mesh,
    scratch_types=[
        pltpu.VMEM((2 * SUPC,), jnp.int32),
        pltpu.VMEM((2 * SUPC,), jnp.int32),
        pltpu.VMEM((2 * SUPC,), jnp.int32),
        pltpu.VMEM((CHUNK, TBL), jnp.float32),
        pltpu.VMEM((CHUNK, TBL), jnp.float32),
        pltpu.VMEM((CHUNK, TBL), jnp.float32),
        pltpu.VMEM((CHUNK, TBL), jnp.float32),
        pltpu.VMEM((CHUNK, IN_DIM), jnp.float32),
        pltpu.VMEM((ATTN_DIM,), jnp.float32),
        pltpu.VMEM((16,), jnp.float32),
        pltpu.VMEM_SHARED((N_NODE, IN_DIM), jnp.float32),
        pltpu.SemaphoreType.DMA,
        pltpu.SemaphoreType.DMA,
        pltpu.SemaphoreType.DMA,
        pltpu.SemaphoreType.DMA,
        pltpu.SemaphoreType.DMA,
        pltpu.SemaphoreType.DMA,
        pltpu.SemaphoreType.DMA,
        pltpu.SemaphoreType.DMA,
    ],
)
def _sc_edges(s_hbm, r_hbm, sub_hbm, eid_hbm, obj_hbm, w_hbm, b_hbm,
              zeros_hbm, out_hbm,
              isub, ieid, iobj, bs0, bs1, br0, br1, msg, wbuf, bbuf, acc,
              sem_s0, sem_s1, sem_r0, sem_r1, sem_ia, sem_ib, sem_ic, sem_sc):
    cid = lax.axis_index("c")
    sid = lax.axis_index("s")
    wid = sid * NC + cid

    r0 = sid * ROWS_PER_TILE
    pltpu.sync_copy(zeros_hbm.at[pl.ds(r0, ROWS_PER_TILE)],
                    acc.at[pl.ds(r0, ROWS_PER_TILE)])

    @pl.when(sid == NS - 1)
    def _zero_tail():
        pltpu.sync_copy(zeros_hbm.at[pl.ds(NS * ROWS_PER_TILE, ROWS_TAIL)],
                        acc.at[pl.ds(NS * ROWS_PER_TILE, ROWS_TAIL)])

    pltpu.sync_copy(w_hbm, wbuf)
    pltpu.sync_copy(b_hbm, bbuf)
    plsc.subcore_barrier()

    base_w = wid * EPW
    bufs = ((bs0, br0, sem_s0, sem_r0), (bs1, br1, sem_s1, sem_r1))

    def load_idx(si):
        off = (si % 2) * SUPC
        base = base_w + si * SUPC
        pltpu.async_copy(sub_hbm.at[pl.ds(base, SUPC)],
                         isub.at[pl.ds(off, SUPC)], sem_ia)
        pltpu.async_copy(eid_hbm.at[pl.ds(base, SUPC)],
                         ieid.at[pl.ds(off, SUPC)], sem_ib)
        pltpu.async_copy(obj_hbm.at[pl.ds(base, SUPC)],
                         iobj.at[pl.ds(off, SUPC)], sem_ic)

    def wait_idx(si):
        off = (si % 2) * SUPC
        base = base_w + si * SUPC
        pltpu.make_async_copy(sub_hbm.at[pl.ds(base, SUPC)],
                              isub.at[pl.ds(off, SUPC)], sem_ia).wait()
        pltpu.make_async_copy(eid_hbm.at[pl.ds(base, SUPC)],
                              ieid.at[pl.ds(off, SUPC)], sem_ib).wait()
        pltpu.make_async_copy(obj_hbm.at[pl.ds(base, SUPC)],
                              iobj.at[pl.ds(off, SUPC)], sem_ic).wait()

    def gather(idx_off, par):
        bs, br, ss, sr = bufs[par]
        pltpu.async_copy(s_hbm.at[isub.at[pl.ds(idx_off, CHUNK)]], bs, ss)
        pltpu.async_copy(r_hbm.at[ieid.at[pl.ds(idx_off, CHUNK)]], br, sr)

    def finish(idx_off, par):
        bs, br, ss, sr = bufs[par]
        pltpu.make_async_copy(s_hbm.at[isub.at[pl.ds(idx_off, CHUNK)]],
                              bs, ss).wait()
        pltpu.make_async_copy(r_hbm.at[ieid.at[pl.ds(idx_off, CHUNK)]],
                              br, sr).wait()

        pltpu.make_async_copy(
            msg, acc.at[iobj.at[pl.ds(idx_off, CHUNK)]], sem_sc).wait()

        w_vecs = [wbuf[pl.ds(16 * j, 16)] for j in range(4)]
        bvec = bbuf[...]

        @plsc.parallel_loop(0, CHUNK, unroll=8)
        def edge_body(e):
            u = None
            for j in range(4):
                pre = bs[e, pl.ds(16 * j, 16)] + br[e, pl.ds(16 * j, 16)]
                t = _rrelu(pre) * w_vecs[j]
                u = t if u is None else u + t
            dot = _hsum16(u)
            av = 1.0 / (1.0 + jnp.exp(-(dot + bvec)))
            for j in range(8):
                m = (bs[e, pl.ds(ATTN_DIM + 16 * j, 16)]
                     + br[e, pl.ds(ATTN_DIM + 16 * j, 16)]) * av
                msg[e, pl.ds(16 * j, 16)] = m

        pltpu.async_copy(msg, acc.at[iobj.at[pl.ds(idx_off, CHUNK)]], sem_sc,
                         add=True)

    load_idx(0)
    wait_idx(0)
    gather(0, 0)
    pltpu.sync_copy(zeros_hbm.at[pl.ds(0, CHUNK)], msg)
    pltpu.async_copy(msg, acc.at[iobj.at[pl.ds(0, CHUNK)]], sem_sc, add=True)

    def super_body(si, carry):
        off = (si % 2) * SUPC

        @pl.when(si < NSUPER - 1)
        def _prefetch_idx():
            load_idx(si + 1)

        def pair_body(j, c2):
            o0 = off + (2 * j) * CHUNK
            gather(o0 + CHUNK, 1)
            finish(o0, 0)

            @pl.when(j < CPS // 2 - 1)
            def _next_even():
                gather(o0 + 2 * CHUNK, 0)

            finish(o0 + CHUNK, 1)
            return c2

        lax.fori_loop(0, CPS // 2, pair_body, 0)

        @pl.when(si < NSUPER - 1)
        def _start_next_super():
            wait_idx(si + 1)
            gather(((si + 1) % 2) * SUPC, 0)

        return carry

    lax.fori_loop(0, NSUPER, super_body, 0)
    pltpu.make_async_copy(
        msg, acc.at[iobj.at[pl.ds(0, CHUNK)]], sem_sc).wait()

    plsc.subcore_barrier()
    pltpu.sync_copy(acc.at[pl.ds(r0, ROWS_PER_TILE)],
                    out_hbm.at[cid, pl.ds(r0, ROWS_PER_TILE)])

    @pl.when(sid == NS - 1)
    def _drain_tail():
        pltpu.sync_copy(acc.at[pl.ds(NS * ROWS_PER_TILE, ROWS_TAIL)],
                        out_hbm.at[cid, pl.ds(NS * ROWS_PER_TILE, ROWS_TAIL)])


def _final_body(a0_ref, a1_ref, wh_ref, o_ref):
    acc = a0_ref[...] + a1_ref[...]
    o_ref[...] = _rrelu(jnp.dot(acc, wh_ref[...],
                                preferred_element_type=jnp.float32,
                                precision=lax.Precision.HIGHEST))


def _final(acc0, acc1, W_h):
    return pl.pallas_call(
        _final_body,
        grid=(10,),
        in_specs=[pl.BlockSpec((1000, IN_DIM), lambda i: (i, 0)),
                  pl.BlockSpec((1000, IN_DIM), lambda i: (i, 0)),
                  pl.BlockSpec((IN_DIM, IN_DIM), lambda i: (0, 0))],
        out_specs=pl.BlockSpec((1000, IN_DIM), lambda i: (i, 0)),
        out_shape=jax.ShapeDtypeStruct((N_NODE, IN_DIM), jnp.float32),
    )(acc0, acc1, W_h)


def kernel(hidden, rel_embeddings, q_rel, batch_idx, rel, sub, obj,
           Ws, Wr, Wqr, bqr, w_alpha, b_alpha, W_h):
    q_idx = q_rel.astype(jnp.int32) + jnp.arange(B, dtype=jnp.int32) * (R + 1)
    onehot = (q_idx[:, None]
              == jnp.arange(B * (R + 1), dtype=jnp.int32)[None, :]
              ).astype(jnp.float32)
    q_sel = _qsel(onehot, rel_embeddings)
    q_rep = jnp.repeat(q_sel, R + 1, axis=0)

    proj_s = _proj_node(hidden, Ws)
    proj_r = _proj_rel(rel_embeddings, q_rep, Wr, Wqr,
                       bqr.reshape(1, ATTN_DIM))

    pad_s = jnp.zeros((N_NODE, TBL - ATTN_DIM - IN_DIM), jnp.float32)
    pad_r = jnp.zeros((B * (R + 1), TBL - ATTN_DIM - IN_DIM), jnp.float32)
    s_tbl = jnp.concatenate([proj_s, hidden, pad_s], axis=1)
    rt_tbl = jnp.concatenate([proj_r, rel_embeddings, pad_r], axis=1)

    eidx = (rel + batch_idx * (R + 1)).astype(jnp.int32)
    acc = _sc_edges(s_tbl, rt_tbl, sub.astype(jnp.int32), eidx,
                    obj.astype(jnp.int32), w_alpha[:, 0],
                    jnp.full((16,), b_alpha[0], jnp.float32),
                    jnp.zeros((N_NODE, IN_DIM), jnp.float32))

    return _final(acc[0], acc[1], W_h)
